# baseline (speedup 1.0000x reference)
import functools

import jax
import jax.numpy as jnp
from jax import lax
from jax.experimental import pallas as pl
from jax.experimental.pallas import tpu as pltpu


def _roll(v, shift):
    n = v.shape[-1]
    return pltpu.roll(v, shift % n, axis=v.ndim - 1)


def _body(mu_ref, sig_ref, x_ref, d_ref, y_ref, loss_ref,
          qb_ref, mi_ref, dsp_ref, acc_ref, sm_ref,
          *, nb, d_dim, kb, s_rows):
    i = pl.program_id(0)
    half = d_dim // 2

    @pl.when(i == 0)
    def _init():
        mu = mu_ref[...]
        sig = sig_ref[...]
        rm = lax.broadcasted_iota(jnp.int32, (s_rows, 128), 0) % 4
        qb_ref[...] = jnp.where(
            rm == 0, jnp.broadcast_to(mu[:, :half], (s_rows, 128)),
            jnp.where(rm == 1, jnp.broadcast_to(sig[:, :half], (s_rows, 128)),
                      jnp.where(rm == 2,
                                jnp.broadcast_to(mu[:, half:], (s_rows, 128)),
                                jnp.broadcast_to(sig[:, half:],
                                                 (s_rows, 128)))))
        lane = lax.broadcasted_iota(jnp.int32, (1, s_rows), 1)
        mi_ref[...] = jnp.where(lane % 4 == 0, 0.0, jnp.inf)
        acc_ref[...] = jnp.zeros_like(acc_ref)
        sm_ref[0] = -jnp.inf
        sm_ref[1] = 0.0

    x = x_ref[...]
    diff = x - qb_ref[...]
    sq = diff * diff
    sqh = sq.astype(jnp.bfloat16).astype(jnp.float32)
    sql = sq - sqh
    ones = jnp.ones((1, 128), jnp.float32)
    dims = (((1,), (1,)), ((), ()))
    drow = (lax.dot_general(ones, sqh, dims,
                            preferred_element_type=jnp.float32)
            + lax.dot_general(ones, sql, dims,
                              preferred_element_type=jnp.float32))
    dsp_ref[i, :] = drow[0, :]
    f = drow + _roll(drow, -1) + _roll(drow, -2) + _roll(drow, -3)
    fv = f + mi_ref[...]

    logits = -fv
    mb = jnp.max(logits)
    m_old = sm_ref[0]
    m_new = jnp.maximum(m_old, mb)
    c = jnp.exp(m_old - m_new)
    w = jnp.exp(logits - m_new).astype(jnp.bfloat16).astype(jnp.float32)
    sm_ref[0] = m_new
    sm_ref[1] = sm_ref[1] * c + jnp.sum(w)
    w4 = jnp.concatenate(
        [w, _roll(w, 1), _roll(w, 2), _roll(w, 3)], axis=0)
    acc_ref[...] = acc_ref[...] * c + lax.dot_general(
        w4, x, (((1,), (0,)), ((), ())),
        preferred_element_type=jnp.float32)

    @pl.when(i == nb - 1)
    def _fin():
        dall = jnp.sum(dsp_ref[...].reshape(nb, kb, 4), axis=2)
        d_ref[...] = dall
        a = acc_ref[...] / sm_ref[1]
        quant = jnp.concatenate([a[0:1, :], a[2:3, :]], axis=1)
        y_ref[...] = quant
        e = quant - mu_ref[...]
        loss_ref[...] = (jnp.sum(e * e) / d_dim).reshape(1, 1)


def kernel(input_mu, input_sig, on_states):
    k_dim, d_dim, _ = on_states.shape
    x = (on_states.transpose(0, 2, 1)
         .reshape(k_dim, 2, 2, d_dim // 2)
         .transpose(0, 2, 1, 3)
         .reshape(4 * k_dim, d_dim // 2))
    mu = input_mu.reshape(1, d_dim)
    sig = input_sig.reshape(1, d_dim)

    kb = 512
    nb = k_dim // kb
    s_rows = 4 * kb

    dists2d, y, loss = pl.pallas_call(
        functools.partial(_body, nb=nb, d_dim=d_dim, kb=kb, s_rows=s_rows),
        grid=(nb,),
        in_specs=[
            pl.BlockSpec((1, d_dim), lambda i: (0, 0)),
            pl.BlockSpec((1, d_dim), lambda i: (0, 0)),
            pl.BlockSpec((s_rows, 128), lambda i: (i, 0)),
        ],
        out_specs=[
            pl.BlockSpec((nb, kb), lambda i: (0, 0)),
            pl.BlockSpec((1, d_dim), lambda i: (0, 0)),
            pl.BlockSpec((1, 1), lambda i: (0, 0)),
        ],
        out_shape=[
            jax.ShapeDtypeStruct((nb, kb), jnp.float32),
            jax.ShapeDtypeStruct((1, d_dim), jnp.float32),
            jax.ShapeDtypeStruct((1, 1), jnp.float32),
        ],
        scratch_shapes=[
            pltpu.VMEM((s_rows, 128), jnp.float32),
            pltpu.VMEM((1, s_rows), jnp.float32),
            pltpu.VMEM((nb, s_rows), jnp.float32),
            pltpu.VMEM((4, 128), jnp.float32),
            pltpu.SMEM((2,), jnp.float32),
        ],
    )(mu, sig, x)

    quantised = y.reshape(d_dim)
    loss_s = loss.reshape(())
    return (quantised, loss_s, loss_s, dists2d.reshape(k_dim))


# speedup vs baseline: 1.3087x; 1.3087x over previous
import functools

import jax
import jax.numpy as jnp
from jax import lax
from jax.experimental import pallas as pl
from jax.experimental.pallas import tpu as pltpu


def _roll(v, shift):
    n = v.shape[-1]
    return pltpu.roll(v, shift % n, axis=v.ndim - 1)


def _body(mu_ref, sig_ref, x_ref, d_ref, y_ref, loss_ref,
          qb_ref, mi_ref, dsp_ref, acc_ref, sm_ref,
          *, nb, d_dim, kb, s_rows):
    i = pl.program_id(0)
    half = d_dim // 2

    @pl.when(i == 0)
    def _init():
        mu = mu_ref[...]
        sig = sig_ref[...]
        rm = lax.broadcasted_iota(jnp.int32, (s_rows, 128), 0) % 4
        qb_ref[...] = jnp.where(
            rm == 0, jnp.broadcast_to(mu[:, :half], (s_rows, 128)),
            jnp.where(rm == 1, jnp.broadcast_to(sig[:, :half], (s_rows, 128)),
                      jnp.where(rm == 2,
                                jnp.broadcast_to(mu[:, half:], (s_rows, 128)),
                                jnp.broadcast_to(sig[:, half:],
                                                 (s_rows, 128)))))
        lane = lax.broadcasted_iota(jnp.int32, (1, s_rows), 1)
        mi_ref[...] = jnp.where(lane % 4 == 0, 0.0, jnp.inf)
        acc_ref[...] = jnp.zeros_like(acc_ref)
        sm_ref[0] = -jnp.inf
        sm_ref[1] = 0.0

    x = x_ref[...]
    diff = x - qb_ref[...]
    sq = diff * diff
    sqh = sq.astype(jnp.bfloat16).astype(jnp.float32)
    sql = sq - sqh
    ones = jnp.ones((1, 128), jnp.float32)
    dims = (((1,), (1,)), ((), ()))
    drow = (lax.dot_general(ones, sqh, dims,
                            preferred_element_type=jnp.float32)
            + lax.dot_general(ones, sql, dims,
                              preferred_element_type=jnp.float32))
    dsp_ref[i, :] = drow[0, :]
    f = drow + _roll(drow, -1) + _roll(drow, -2) + _roll(drow, -3)
    fv = f + mi_ref[...]

    logits = -fv
    mb = jnp.max(logits)
    m_old = sm_ref[0]
    m_new = jnp.maximum(m_old, mb)
    c = jnp.exp(m_old - m_new)
    w = jnp.exp(logits - m_new).astype(jnp.bfloat16).astype(jnp.float32)
    sm_ref[0] = m_new
    sm_ref[1] = sm_ref[1] * c + jnp.sum(w)
    w4 = jnp.concatenate(
        [w, _roll(w, 1), _roll(w, 2), _roll(w, 3)], axis=0)
    acc_ref[...] = acc_ref[...] * c + lax.dot_general(
        w4, x, (((1,), (0,)), ((), ())),
        preferred_element_type=jnp.float32)

    @pl.when(i == nb - 1)
    def _fin():
        dall = jnp.sum(dsp_ref[...].reshape(nb, kb, 4), axis=2)
        d_ref[...] = dall
        a = acc_ref[...] / sm_ref[1]
        quant = jnp.concatenate([a[0:1, :], a[2:3, :]], axis=1)
        y_ref[...] = quant
        e = quant - mu_ref[...]
        loss_ref[...] = (jnp.sum(e * e) / d_dim).reshape(1, 1)


def kernel(input_mu, input_sig, on_states):
    k_dim, d_dim, _ = on_states.shape
    x = (on_states.transpose(0, 2, 1)
         .reshape(k_dim, 2, 2, d_dim // 2)
         .transpose(0, 2, 1, 3)
         .reshape(4 * k_dim, d_dim // 2))
    mu = input_mu.reshape(1, d_dim)
    sig = input_sig.reshape(1, d_dim)

    kb = 2048
    nb = k_dim // kb
    s_rows = 4 * kb

    dists2d, y, loss = pl.pallas_call(
        functools.partial(_body, nb=nb, d_dim=d_dim, kb=kb, s_rows=s_rows),
        grid=(nb,),
        in_specs=[
            pl.BlockSpec((1, d_dim), lambda i: (0, 0)),
            pl.BlockSpec((1, d_dim), lambda i: (0, 0)),
            pl.BlockSpec((s_rows, 128), lambda i: (i, 0)),
        ],
        out_specs=[
            pl.BlockSpec((nb, kb), lambda i: (0, 0)),
            pl.BlockSpec((1, d_dim), lambda i: (0, 0)),
            pl.BlockSpec((1, 1), lambda i: (0, 0)),
        ],
        out_shape=[
            jax.ShapeDtypeStruct((nb, kb), jnp.float32),
            jax.ShapeDtypeStruct((1, d_dim), jnp.float32),
            jax.ShapeDtypeStruct((1, 1), jnp.float32),
        ],
        scratch_shapes=[
            pltpu.VMEM((s_rows, 128), jnp.float32),
            pltpu.VMEM((1, s_rows), jnp.float32),
            pltpu.VMEM((nb, s_rows), jnp.float32),
            pltpu.VMEM((4, 128), jnp.float32),
            pltpu.SMEM((2,), jnp.float32),
        ],
    )(mu, sig, x)

    quantised = y.reshape(d_dim)
    loss_s = loss.reshape(())
    return (quantised, loss_s, loss_s, dists2d.reshape(k_dim))
